# Initial kernel scaffold; baseline (speedup 1.0000x reference)
import jax
import jax.numpy as jnp
from jax.experimental import pallas as pl


def kernel(emotion_ids, embedding_weight):
    raise NotImplementedError("write your pallas kernel here")


# speedup vs baseline: 1.4714x; 1.4714x over previous
import functools

import jax
import jax.numpy as jnp
from jax import lax
from jax.experimental import pallas as pl
from jax.experimental.pallas import tpu as pltpu
from jax.experimental.pallas import tpu_sc as plsc

_IDX_CHUNK = 128


def _pair_gather(pair_ids_2d, pair_table, npairs):
    total_chunks, idx_chunk = pair_ids_2d.shape
    _, dim2 = pair_table.shape
    info = plsc.get_sparse_core_info()
    nw = info.num_cores * info.num_subcores
    p_per_w = npairs // nw
    n_chunks = total_chunks // nw

    mesh = plsc.VectorSubcoreMesh(core_axis_name="c", subcore_axis_name="s")

    @functools.partial(
        pl.kernel,
        mesh=mesh,
        out_type=jax.ShapeDtypeStruct((npairs, dim2), jnp.float32),
        scratch_types=[
            pltpu.VMEM((n_chunks, idx_chunk), jnp.int32),
            pltpu.VMEM((p_per_w, dim2), jnp.float32),
            pltpu.SemaphoreType.DMA,
        ],
    )
    def emb(idx_hbm, table_hbm, out_hbm, idx_v, rows_v, sem):
        wid = lax.axis_index("s") * info.num_cores + lax.axis_index("c")
        pltpu.sync_copy(idx_hbm.at[pl.ds(wid * n_chunks, n_chunks)], idx_v)
        copies = [
            pltpu.async_copy(
                table_hbm.at[idx_v.at[k]],
                rows_v.at[pl.ds(k * idx_chunk, idx_chunk)],
                sem,
            )
            for k in range(n_chunks)
        ]
        for c in copies:
            c.wait()
        pltpu.sync_copy(rows_v, out_hbm.at[pl.ds(wid * p_per_w, p_per_w)])

    return emb(pair_ids_2d, pair_table)


def kernel(emotion_ids, embedding_weight):
    ids = emotion_ids.astype(jnp.int32)
    table = embedding_weight.astype(jnp.float32)
    vocab, dim = table.shape
    batch, = ids.shape

    left = jnp.repeat(table, vocab, axis=0)
    right = jnp.tile(table, (vocab, 1))
    pair_table = jnp.concatenate([left, right], axis=1)

    idp = ids.reshape(batch // 2, 2)
    pair_ids = idp[:, 0] * vocab + idp[:, 1]
    pair_ids_2d = pair_ids.reshape(-1, _IDX_CHUNK)

    out2 = _pair_gather(pair_ids_2d, pair_table, batch // 2)
    return out2.reshape(batch, dim)
